# Initial kernel scaffold; baseline (speedup 1.0000x reference)
import jax
import jax.numpy as jnp
from jax.experimental import pallas as pl


def kernel(x, adj, weight, att_self_weight, att_neighs_weight, bias_weight):
    raise NotImplementedError("write your pallas kernel here")


# speedup vs baseline: 1.9804x; 1.9804x over previous
import functools

import jax
import jax.numpy as jnp
import numpy as np
from jax.experimental import pallas as pl

_N = 2048
_F = 512
_H = 8
_E = 64
_BLK = 256


def _feat_scores_kernel(x_ref, w_ref, ws_ref, feat_ref, scores_ref):
    feat = jnp.dot(x_ref[...], w_ref[...], preferred_element_type=jnp.float32)
    feat_ref[...] = feat
    scores_ref[...] = jnp.dot(feat, ws_ref[...], preferred_element_type=jnp.float32)


def _attn_kernel(sself_ref, sneighT_ref, adj_ref, feat_ref, bias_ref, out_ref):
    mask = -1e9 * (1.0 - adj_ref[...])
    for h in range(_H):
        a_i = sself_ref[:, h : h + 1]
        a_j = sneighT_ref[h : h + 1, :]
        s = a_i + a_j
        s = jnp.where(s > 0, s, 0.2 * s)
        s = s + mask
        m = jnp.max(s, axis=-1, keepdims=True)
        e = jnp.exp(s - m)
        denom = jnp.sum(e, axis=-1, keepdims=True)
        v = feat_ref[:, h * _E : (h + 1) * _E]
        o = jnp.dot(e, v, preferred_element_type=jnp.float32)
        o = o / denom + bias_ref[:, h * _E : (h + 1) * _E]
        out_ref[:, h * _E : (h + 1) * _E] = jnp.maximum(o, 0.0)


@functools.partial(jax.jit, static_argnames=())
def kernel(x, adj, weight, att_self_weight, att_neighs_weight, bias_weight):
    n_blocks = _N // _BLK

    eye = jnp.eye(_H, dtype=jnp.float32)
    sel = jnp.repeat(eye, _E, axis=0)
    ws = jnp.concatenate(
        [sel * att_self_weight.reshape(_H * _E, 1),
         sel * att_neighs_weight.reshape(_H * _E, 1)], axis=1)

    feat, scores = pl.pallas_call(
        _feat_scores_kernel,
        grid=(n_blocks,),
        in_specs=[
            pl.BlockSpec((_BLK, _F), lambda i: (i, 0)),
            pl.BlockSpec((_F, _H * _E), lambda i: (0, 0)),
            pl.BlockSpec((_F, 2 * _H), lambda i: (0, 0)),
        ],
        out_specs=[
            pl.BlockSpec((_BLK, _H * _E), lambda i: (i, 0)),
            pl.BlockSpec((_BLK, 2 * _H), lambda i: (i, 0)),
        ],
        out_shape=[
            jax.ShapeDtypeStruct((_N, _H * _E), jnp.float32),
            jax.ShapeDtypeStruct((_N, 2 * _H), jnp.float32),
        ],
    )(x, weight, ws)

    sself = scores[:, :_H]
    sneighT = scores[:, _H:].T
    bias2d = bias_weight.reshape(1, _H * _E)

    out = pl.pallas_call(
        _attn_kernel,
        grid=(n_blocks,),
        in_specs=[
            pl.BlockSpec((_BLK, _H), lambda i: (i, 0)),
            pl.BlockSpec((_H, _N), lambda i: (0, 0)),
            pl.BlockSpec((_BLK, _N), lambda i: (i, 0)),
            pl.BlockSpec((_N, _H * _E), lambda i: (0, 0)),
            pl.BlockSpec((1, _H * _E), lambda i: (0, 0)),
        ],
        out_specs=pl.BlockSpec((_BLK, _H * _E), lambda i: (i, 0)),
        out_shape=jax.ShapeDtypeStruct((_N, _H * _E), jnp.float32),
    )(sself, sneighT, adj, feat, bias2d)

    return out


# speedup vs baseline: 2.5516x; 1.2884x over previous
import functools

import jax
import jax.numpy as jnp
import numpy as np
from jax.experimental import pallas as pl

_N = 2048
_F = 512
_H = 8
_E = 64
_BLK = 256


def _feat_scores_kernel(x_ref, w_ref, ws_ref, feat_ref, scores_ref):
    feat = jnp.dot(x_ref[...], w_ref[...], preferred_element_type=jnp.float32)
    feat_ref[...] = feat
    scores_ref[...] = jnp.dot(feat, ws_ref[...], preferred_element_type=jnp.float32)


def _attn_kernel(sself_ref, sneighT_ref, adj_ref, feat_ref, bias_ref, out_ref):
    adj = adj_ref[...]
    for h in range(_H):
        a_i = sself_ref[:, h : h + 1]
        a_j = sneighT_ref[h : h + 1, :]
        s = a_i + a_j
        s = jnp.maximum(s, 0.2 * s)
        e = adj * jnp.exp(s)
        denom = jnp.sum(e, axis=-1, keepdims=True)
        v = feat_ref[:, h * _E : (h + 1) * _E]
        o = jnp.dot(e, v, preferred_element_type=jnp.float32)
        o = o / denom + bias_ref[:, h * _E : (h + 1) * _E]
        out_ref[:, h * _E : (h + 1) * _E] = jnp.maximum(o, 0.0)


@functools.partial(jax.jit, static_argnames=())
def kernel(x, adj, weight, att_self_weight, att_neighs_weight, bias_weight):
    n_blocks = _N // _BLK

    eye = jnp.eye(_H, dtype=jnp.float32)
    sel = jnp.repeat(eye, _E, axis=0)
    ws = jnp.concatenate(
        [sel * att_self_weight.reshape(_H * _E, 1),
         sel * att_neighs_weight.reshape(_H * _E, 1)], axis=1)

    feat, scores = pl.pallas_call(
        _feat_scores_kernel,
        grid=(n_blocks,),
        in_specs=[
            pl.BlockSpec((_BLK, _F), lambda i: (i, 0)),
            pl.BlockSpec((_F, _H * _E), lambda i: (0, 0)),
            pl.BlockSpec((_F, 2 * _H), lambda i: (0, 0)),
        ],
        out_specs=[
            pl.BlockSpec((_BLK, _H * _E), lambda i: (i, 0)),
            pl.BlockSpec((_BLK, 2 * _H), lambda i: (i, 0)),
        ],
        out_shape=[
            jax.ShapeDtypeStruct((_N, _H * _E), jnp.float32),
            jax.ShapeDtypeStruct((_N, 2 * _H), jnp.float32),
        ],
    )(x, weight, ws)

    sself = scores[:, :_H]
    sneighT = scores[:, _H:].T
    bias2d = bias_weight.reshape(1, _H * _E)

    out = pl.pallas_call(
        _attn_kernel,
        grid=(n_blocks,),
        in_specs=[
            pl.BlockSpec((_BLK, _H), lambda i: (i, 0)),
            pl.BlockSpec((_H, _N), lambda i: (0, 0)),
            pl.BlockSpec((_BLK, _N), lambda i: (i, 0)),
            pl.BlockSpec((_N, _H * _E), lambda i: (0, 0)),
            pl.BlockSpec((1, _H * _E), lambda i: (0, 0)),
        ],
        out_specs=pl.BlockSpec((_BLK, _H * _E), lambda i: (i, 0)),
        out_shape=jax.ShapeDtypeStruct((_N, _H * _E), jnp.float32),
    )(sself, sneighT, adj, feat, bias2d)

    return out


# speedup vs baseline: 3.2094x; 1.2578x over previous
import jax
import jax.numpy as jnp
from jax.experimental import pallas as pl

_N = 2048
_F = 512
_H = 8
_E = 64
_VW = 128
_BLK = 256


def _feat_scores_kernel(x_ref, w_ref, ws_ref, vaug_ref, scores_ref):
    feat = jnp.dot(x_ref[...], w_ref[...], preferred_element_type=jnp.float32)
    scores_ref[...] = jnp.dot(feat, ws_ref[...], preferred_element_type=jnp.float32)
    ones_pad = jnp.where(
        jax.lax.broadcasted_iota(jnp.int32, (_BLK, _VW - _E), 1) == 0, 1.0, 0.0
    )
    for h in range(_H):
        vaug_ref[:, h * _VW : h * _VW + _E] = feat[:, h * _E : (h + 1) * _E]
        vaug_ref[:, h * _VW + _E : (h + 1) * _VW] = ones_pad


def _attn_kernel(sself_ref, sneighT_ref, adj_ref, vaug_ref, bias_ref, out_ref):
    adj = adj_ref[...]
    for h in range(_H):
        a_i = sself_ref[:, h : h + 1]
        a_j = sneighT_ref[h : h + 1, :]
        p, p2 = jnp.exp(a_i), jnp.exp(0.2 * a_i)
        q, q2 = jnp.exp(a_j), jnp.exp(0.2 * a_j)
        e = adj * jnp.maximum(p * q, p2 * q2)
        o2 = jnp.dot(
            e, vaug_ref[:, h * _VW : (h + 1) * _VW],
            preferred_element_type=jnp.float32,
        )
        o = o2[:, :_E] / o2[:, _E : _E + 1] + bias_ref[:, h * _E : (h + 1) * _E]
        out_ref[:, h * _E : (h + 1) * _E] = jnp.maximum(o, 0.0)


def kernel(x, adj, weight, att_self_weight, att_neighs_weight, bias_weight):
    n_blocks = _N // _BLK

    eye = jnp.eye(_H, dtype=jnp.float32)
    sel = jnp.repeat(eye, _E, axis=0)
    ws = jnp.concatenate(
        [sel * att_self_weight.reshape(_H * _E, 1),
         sel * att_neighs_weight.reshape(_H * _E, 1)], axis=1)

    vaug, scores = pl.pallas_call(
        _feat_scores_kernel,
        grid=(n_blocks,),
        in_specs=[
            pl.BlockSpec((_BLK, _F), lambda i: (i, 0)),
            pl.BlockSpec((_F, _H * _E), lambda i: (0, 0)),
            pl.BlockSpec((_H * _E, 2 * _H), lambda i: (0, 0)),
        ],
        out_specs=[
            pl.BlockSpec((_BLK, _H * _VW), lambda i: (i, 0)),
            pl.BlockSpec((_BLK, 2 * _H), lambda i: (i, 0)),
        ],
        out_shape=[
            jax.ShapeDtypeStruct((_N, _H * _VW), jnp.float32),
            jax.ShapeDtypeStruct((_N, 2 * _H), jnp.float32),
        ],
    )(x, weight, ws)

    sself = scores[:, :_H]
    sneighT = scores[:, _H:].T
    bias2d = bias_weight.reshape(1, _H * _E)

    out = pl.pallas_call(
        _attn_kernel,
        grid=(n_blocks,),
        in_specs=[
            pl.BlockSpec((_BLK, _H), lambda i: (i, 0)),
            pl.BlockSpec((_H, _N), lambda i: (0, 0)),
            pl.BlockSpec((_BLK, _N), lambda i: (i, 0)),
            pl.BlockSpec((_N, _H * _VW), lambda i: (0, 0)),
            pl.BlockSpec((1, _H * _E), lambda i: (0, 0)),
        ],
        out_specs=pl.BlockSpec((_BLK, _H * _E), lambda i: (i, 0)),
        out_shape=jax.ShapeDtypeStruct((_N, _H * _E), jnp.float32),
    )(sself, sneighT, adj, vaug, bias2d)

    return out


# speedup vs baseline: 3.4757x; 1.0830x over previous
import jax
import jax.numpy as jnp
from jax.experimental import pallas as pl

_N = 2048
_F = 512
_H = 8
_E = 64
_VW = 128
_BLK = 256


def _feat_scores_kernel(x_ref, w_ref, ws_ref, vaug_ref, scores_ref):
    feat = jnp.dot(x_ref[...], w_ref[...], preferred_element_type=jnp.float32)
    scores_ref[...] = jnp.dot(feat, ws_ref[...], preferred_element_type=jnp.float32)
    ones_pad = jnp.where(
        jax.lax.broadcasted_iota(jnp.int32, (_BLK, _VW - _E), 1) == 0, 1.0, 0.0
    ).astype(jnp.bfloat16)
    feat16 = feat.astype(jnp.bfloat16)
    for h in range(_H):
        vaug_ref[:, h * _VW : h * _VW + _E] = feat16[:, h * _E : (h + 1) * _E]
        vaug_ref[:, h * _VW + _E : (h + 1) * _VW] = ones_pad


def _attn_kernel(sself_ref, sneighT_ref, adj_ref, vaug_ref, bias_ref, out_ref):
    adj = adj_ref[...].astype(jnp.bfloat16)
    for h in range(_H):
        a_i = sself_ref[:, h : h + 1]
        a_j = sneighT_ref[h : h + 1, :]
        p = jnp.exp(a_i).astype(jnp.bfloat16)
        p2 = jnp.exp(0.2 * a_i).astype(jnp.bfloat16)
        q = jnp.exp(a_j).astype(jnp.bfloat16)
        q2 = jnp.exp(0.2 * a_j).astype(jnp.bfloat16)
        e = adj * jnp.maximum(p * q, p2 * q2)
        o2 = jnp.dot(
            e, vaug_ref[:, h * _VW : (h + 1) * _VW],
            preferred_element_type=jnp.float32,
        )
        o = o2[:, :_E] / o2[:, _E : _E + 1] + bias_ref[:, h * _E : (h + 1) * _E]
        out_ref[:, h * _E : (h + 1) * _E] = jnp.maximum(o, 0.0)


def kernel(x, adj, weight, att_self_weight, att_neighs_weight, bias_weight):
    n_blocks = _N // _BLK

    eye = jnp.eye(_H, dtype=jnp.float32)
    sel = jnp.repeat(eye, _E, axis=0)
    ws = jnp.concatenate(
        [sel * att_self_weight.reshape(_H * _E, 1),
         sel * att_neighs_weight.reshape(_H * _E, 1)], axis=1)

    vaug, scores = pl.pallas_call(
        _feat_scores_kernel,
        grid=(n_blocks,),
        in_specs=[
            pl.BlockSpec((_BLK, _F), lambda i: (i, 0)),
            pl.BlockSpec((_F, _H * _E), lambda i: (0, 0)),
            pl.BlockSpec((_H * _E, 2 * _H), lambda i: (0, 0)),
        ],
        out_specs=[
            pl.BlockSpec((_BLK, _H * _VW), lambda i: (i, 0)),
            pl.BlockSpec((_BLK, 2 * _H), lambda i: (i, 0)),
        ],
        out_shape=[
            jax.ShapeDtypeStruct((_N, _H * _VW), jnp.bfloat16),
            jax.ShapeDtypeStruct((_N, 2 * _H), jnp.float32),
        ],
    )(x, weight, ws)

    sself = scores[:, :_H]
    sneighT = scores[:, _H:].T
    bias2d = bias_weight.reshape(1, _H * _E)

    out = pl.pallas_call(
        _attn_kernel,
        grid=(n_blocks,),
        in_specs=[
            pl.BlockSpec((_BLK, _H), lambda i: (i, 0)),
            pl.BlockSpec((_H, _N), lambda i: (0, 0)),
            pl.BlockSpec((_BLK, _N), lambda i: (i, 0)),
            pl.BlockSpec((_N, _H * _VW), lambda i: (0, 0)),
            pl.BlockSpec((1, _H * _E), lambda i: (0, 0)),
        ],
        out_specs=pl.BlockSpec((_BLK, _H * _E), lambda i: (i, 0)),
        out_shape=jax.ShapeDtypeStruct((_N, _H * _E), jnp.float32),
    )(sself, sneighT, adj, vaug, bias2d)

    return out
